# baseline (speedup 1.0000x reference)
import functools

import jax
import jax.numpy as jnp
from jax import lax
from jax.experimental import pallas as pl
from jax.experimental.pallas import tpu as pltpu
from jax.experimental.pallas import tpu_sc as plsc

_B = 16384
_P = 10
_D = 64
_NW = 32

_POI_PER_W = _B // _NW
_CBG_TOT = 6 * _B
_CBG_PER_W = _CBG_TOT // _NW
_CHUNK = 1024

_BLK = 2048
_GRID = _B // _BLK


_W = 512
_PC = 32768
_PR = _PC // 2
_PCS = _PC.bit_length() - 1

_CBG_GRID = (1000000 + _PC - 1) // _PC
_POI_GRID = (100000 + _PC - 1) // _PC


def _pair_body(tin_ref, out_ref):
    x = tin_ref[...]
    eye = (lax.broadcasted_iota(jnp.int32, (_D, _D), 0)
           == lax.broadcasted_iota(jnp.int32, (_D, _D), 1)).astype(jnp.float32)
    xt = jax.lax.dot_general(x, eye, (((0,), (0,)), ((), ())),
                             preferred_element_type=jnp.float32)
    out_ref[...] = jnp.concatenate([xt[:_PR], xt[_PR:]], axis=1)


def _make_pair(grid):
    return pl.pallas_call(
        _pair_body,
        grid=(grid,),
        in_specs=[pl.BlockSpec((_D, _PC), lambda i: (0, i))],
        out_specs=pl.BlockSpec((_PR, 2 * _D), lambda i: (i, 0)),
        out_shape=jax.ShapeDtypeStruct((grid * _PR, 2 * _D), jnp.float32),
        compiler_params=pltpu.CompilerParams(
            dimension_semantics=("parallel",)),
    )


def _sg_block(table, idx_hbm, out_hbm, base, idxv, rows, sem):
    pltpu.sync_copy(idx_hbm.at[pl.ds(base, _W)], idxv)
    pltpu.async_copy(table.at[idxv], rows, sem).wait()
    pltpu.sync_copy(rows, out_hbm.at[pl.ds(base, _W)])


def _gather_poi_body(poi_pair, poi_idx, poi_out, idxv, rows, sem):
    wid = lax.axis_index("s") * 2 + lax.axis_index("c")
    _sg_block(poi_pair, poi_idx, poi_out, wid * _W, idxv, rows, sem)


_HB = _B // 2
_W2 = _HB // _NW


def _sg_block2(table, idx_hbm, out_hbm, in_base, out_base, idxv, rows, sem):
    pltpu.sync_copy(idx_hbm.at[pl.ds(in_base, _W2)], idxv)
    pltpu.async_copy(table.at[idxv], rows, sem).wait()
    pltpu.sync_copy(rows, out_hbm.at[pl.ds(out_base, _W2)])


def _make_cbg_half_body(h):
    def body(cbg_pair, cbg_idx, cbg_out, idxv, rows, sem):
        wid = lax.axis_index("s") * 2 + lax.axis_index("c")
        for j in range(6):
            _sg_block2(cbg_pair, cbg_idx, cbg_out,
                       j * _B + h * _HB + wid * _W2,
                       j * _HB + wid * _W2, idxv, rows, sem)
    return body


@functools.cache
def _gather_sc():
    mesh = plsc.VectorSubcoreMesh(core_axis_name="c", subcore_axis_name="s")
    params = pltpu.CompilerParams(use_tc_tiling_on_sc=True)
    poi_k = pl.kernel(
        _gather_poi_body, mesh=mesh,
        out_type=[jax.ShapeDtypeStruct((_B, 2 * _D), jnp.float32)],
        scratch_types=[
            pltpu.VMEM((_W,), jnp.int32),
            pltpu.VMEM((_W, 2 * _D), jnp.float32),
            pltpu.SemaphoreType.DMA,
        ],
        compiler_params=params)
    cbg_ks = tuple(
        pl.kernel(
            _make_cbg_half_body(h), mesh=mesh,
            out_type=[jax.ShapeDtypeStruct((6 * _HB, 2 * _D), jnp.float32)],
            scratch_types=[
                pltpu.VMEM((_W2,), jnp.int32),
                pltpu.VMEM((_W2, 2 * _D), jnp.float32),
                pltpu.SemaphoreType.DMA,
            ],
            compiler_params=params)
        for h in range(2))
    return poi_k, cbg_ks


def _log_sigmoid(t):
    return jnp.minimum(t, 0.0) - jnp.log(1.0 + jnp.exp(-jnp.abs(t)))


def _mm(a, b):
    return jax.lax.dot_general(a, b, (((1,), (0,)), ((), ())),
                               preferred_element_type=jnp.float32)


def _mm_t(a, b):
    return jax.lax.dot_general(a, b, (((1,), (1,)), ((), ())),
                               preferred_element_type=jnp.float32)


def _loss_body(x_ref, cate_emb_ref, perc_emb_ref, poi_ref, cbg_ref, par_ref,
               out_ref):
    f32 = jnp.float32
    x = x_ref[...]
    par = par_ref[...]

    def half(xfull, p):
        return jnp.where(p > 0.5, xfull[:, _D:2 * _D], xfull[:, 0:_D])

    r18 = lax.broadcasted_iota(jnp.int32, (18, 5), 0)
    c18 = lax.broadcasted_iota(jnp.int32, (18, 5), 1)
    e_obs = (r18 == 3 + c18).astype(f32)
    e_alt = (r18 == 9 + 2 * c18).astype(f32)
    r5 = lax.broadcasted_iota(jnp.int32, (5, 5 * _P), 0)
    c5 = lax.broadcasted_iota(jnp.int32, (5, 5 * _P), 1)
    rep = (c5 // _P == r5).astype(f32)
    s50 =(lax.broadcasted_iota(jnp.int32, (5 * _P, 5), 0) // _P
           == lax.broadcasted_iota(jnp.int32, (5 * _P, 5), 1)).astype(f32)
    percs = (1.0 / (2.0 * _P)
             + (lax.broadcasted_iota(jnp.int32, (1, 5 * _P), 1) % _P
                ).astype(f32) / _P)
    ones_d = jnp.ones((_D, 1), f32)

    cate_col = x[:, 0:1]
    oh = (cate_col == lax.broadcasted_iota(jnp.int32, (1, 4), 1).astype(f32)
          ).astype(f32)
    cate_e = _mm(oh, cate_emb_ref[...])
    poi_e = half(poi_ref[...], par[:, 0:1])

    def softmax50(fv5):
        lg = -jnp.abs(_P * (_mm(fv5, rep) - percs))
        e = jnp.exp(lg)
        den = _mm(e, s50)
        return e * _mm(1.0 / den, rep)

    m_obs = softmax50(_mm(x, e_obs))
    m_alt = softmax50(_mm(x, e_alt))

    g_cate = _mm_t(perc_emb_ref[...], cate_emb_ref[...])
    gc_sel = _mm_t(oh, g_cate)
    gp = _mm_t(poi_e, perc_emb_ref[...])
    t_obs_c = _mm(m_obs * gc_sel, s50)
    t_obs_p = _mm(m_obs * gp, s50)
    t_alt_c = _mm(m_alt * gc_sel, s50)
    t_alt_p = _mm(m_alt * gp, s50)

    dots = []
    for j in range(6):
        c = half(cbg_ref[j], par[:, j + 1:j + 2])
        sgn = 1.0 if j == 0 else -1.0
        dots.append(_mm(cate_e * c, ones_d) * sgn)
        dots.append(_mm(poi_e * c, ones_d) * sgn)
    packed = jnp.concatenate(
        dots + [t_obs_c, t_obs_p, -t_alt_c, -t_alt_p], axis=1)

    col = lax.broadcasted_iota(jnp.int32, (1, 32), 1)
    w = jnp.where((col >= 2) & (col < 12), 0.2, 1.0)
    total = -jnp.sum(w * _log_sigmoid(packed))

    @pl.when(pl.program_id(0) == 0)
    def _init():
        out_ref[0, 0] = 0.0

    out_ref[0, 0] += total


def _make_loss(h):
    hoff = h * (_HB // _BLK)
    return pl.pallas_call(
        _loss_body,
        grid=(_HB // _BLK,),
        in_specs=[
            pl.BlockSpec((_BLK, 18), lambda i: (i + hoff, 0)),
            pl.BlockSpec((4, _D), lambda i: (0, 0)),
            pl.BlockSpec((5 * _P, _D), lambda i: (0, 0)),
            pl.BlockSpec((_BLK, 2 * _D), lambda i: (i + hoff, 0)),
            pl.BlockSpec((6, _BLK, 2 * _D), lambda i: (0, i, 0)),
            pl.BlockSpec((_BLK, 8), lambda i: (i + hoff, 0)),
        ],
        out_specs=pl.BlockSpec(
            (1, 1), lambda i: (0, 0), memory_space=pltpu.SMEM),
        out_shape=jax.ShapeDtypeStruct((1, 1), jnp.float32),
        compiler_params=pltpu.CompilerParams(
            dimension_semantics=("arbitrary",)),
    )


def kernel(inputs, cate_emb, poi_emb, cbg_emb, perc_emb):
    poi_ids = inputs[:, 1].astype(jnp.int32)
    cbg_cols = [2, 8, 10, 12, 14, 16]
    cbg_ids = jnp.concatenate(
        [inputs[:, c] for c in cbg_cols]).astype(jnp.int32)

    def rowof(i):
        return (i >> _PCS) * _PR + (i & (_PR - 1))

    def parof(i):
        return ((i >> (_PCS - 1)) & 1).astype(jnp.float32)

    poi_k, cbg_ks = _gather_sc()
    cbg_pair = _make_pair(_CBG_GRID)(cbg_emb.T)
    cbg_row_idx = rowof(cbg_ids)
    (cbg_rows_a,) = cbg_ks[0](cbg_pair, cbg_row_idx)
    poi_pair = _make_pair(_POI_GRID)(poi_emb.T)
    (poi_rows,) = poi_k(poi_pair, rowof(poi_ids))
    (cbg_rows_b,) = cbg_ks[1](cbg_pair, cbg_row_idx)
    par = jnp.stack(
        [parof(poi_ids)] + [parof(cbg_ids[j * _B:(j + 1) * _B])
                            for j in range(6)]
        + [jnp.zeros((_B,), jnp.float32)], axis=1)
    out_a = _make_loss(0)(inputs, cate_emb, perc_emb, poi_rows,
                          cbg_rows_a.reshape(6, _HB, 2 * _D), par)
    out_b = _make_loss(1)(inputs, cate_emb, perc_emb, poi_rows,
                          cbg_rows_b.reshape(6, _HB, 2 * _D), par)
    return out_a[0, 0] + out_b[0, 0]


# speedup vs baseline: 1.0048x; 1.0048x over previous
import functools

import jax
import jax.numpy as jnp
from jax import lax
from jax.experimental import pallas as pl
from jax.experimental.pallas import tpu as pltpu
from jax.experimental.pallas import tpu_sc as plsc

_B = 16384
_P = 10
_D = 64
_NW = 32

_POI_PER_W = _B // _NW
_CBG_TOT = 6 * _B
_CBG_PER_W = _CBG_TOT // _NW
_CHUNK = 1024

_BLK = 2048
_GRID = _B // _BLK


_W = 512
_PC = 32768
_PR = _PC // 2
_PCS = _PC.bit_length() - 1

_CBG_GRID = (1000000 + _PC - 1) // _PC
_POI_GRID = (100000 + _PC - 1) // _PC


def _pair_body(tin_ref, out_ref):
    x = tin_ref[...]
    eye = (lax.broadcasted_iota(jnp.int32, (_D, _D), 0)
           == lax.broadcasted_iota(jnp.int32, (_D, _D), 1)).astype(jnp.float32)
    xt = jax.lax.dot_general(x, eye, (((0,), (0,)), ((), ())),
                             preferred_element_type=jnp.float32)
    out_ref[...] = jnp.concatenate([xt[:_PR], xt[_PR:]], axis=1)


def _make_pair(grid):
    return pl.pallas_call(
        _pair_body,
        grid=(grid,),
        in_specs=[pl.BlockSpec((_D, _PC), lambda i: (0, i))],
        out_specs=pl.BlockSpec((_PR, 2 * _D), lambda i: (i, 0)),
        out_shape=jax.ShapeDtypeStruct((grid * _PR, 2 * _D), jnp.float32),
        compiler_params=pltpu.CompilerParams(
            dimension_semantics=("parallel",)),
    )


def _sg_block(table, idx_hbm, out_hbm, base, idxv, rows, sem):
    pltpu.sync_copy(idx_hbm.at[pl.ds(base, _W)], idxv)
    pltpu.async_copy(table.at[idxv], rows, sem).wait()
    pltpu.sync_copy(rows, out_hbm.at[pl.ds(base, _W)])


def _gather_poi_body(poi_pair, poi_idx, poi_out, idxv, rows, sem):
    wid = lax.axis_index("s") * 2 + lax.axis_index("c")
    _sg_block(poi_pair, poi_idx, poi_out, wid * _W, idxv, rows, sem)


def _gather_cbg_body(cbg_pair, cbg_idx, cbg_out, idxv, rows, sem):
    wid = lax.axis_index("s") * 2 + lax.axis_index("c")
    for j in range(6):
        _sg_block(cbg_pair, cbg_idx, cbg_out, j * _B + wid * _W,
                  idxv, rows, sem)


@functools.cache
def _gather_sc():
    scratch = [
        pltpu.VMEM((_W,), jnp.int32),
        pltpu.VMEM((_W, 2 * _D), jnp.float32),
        pltpu.SemaphoreType.DMA,
    ]
    mesh = plsc.VectorSubcoreMesh(core_axis_name="c", subcore_axis_name="s")
    params = pltpu.CompilerParams(use_tc_tiling_on_sc=True)
    poi_k = pl.kernel(
        _gather_poi_body, mesh=mesh,
        out_type=[jax.ShapeDtypeStruct((_B, 2 * _D), jnp.float32)],
        scratch_types=scratch, compiler_params=params)
    cbg_k = pl.kernel(
        _gather_cbg_body, mesh=mesh,
        out_type=[jax.ShapeDtypeStruct((_CBG_TOT, 2 * _D), jnp.float32)],
        scratch_types=scratch, compiler_params=params)
    return poi_k, cbg_k


def _log_sigmoid(t):
    return jnp.minimum(t, 0.0) - jnp.log(1.0 + jnp.exp(-jnp.abs(t)))


def _mm(a, b):
    return jax.lax.dot_general(a, b, (((1,), (0,)), ((), ())),
                               preferred_element_type=jnp.float32)


def _mm_t(a, b):
    return jax.lax.dot_general(a, b, (((1,), (1,)), ((), ())),
                               preferred_element_type=jnp.float32)


def _loss_body(x_ref, cate_emb_ref, perc_emb_ref, poi_ref, cbg_ref, par_ref,
               out_ref):
    f32 = jnp.float32
    x = x_ref[...]
    par = par_ref[...]

    def half(xfull, p):
        return jnp.where(p > 0.5, xfull[:, _D:2 * _D], xfull[:, 0:_D])

    r18 = lax.broadcasted_iota(jnp.int32, (18, 5), 0)
    c18 = lax.broadcasted_iota(jnp.int32, (18, 5), 1)
    e_obs = (r18 == 3 + c18).astype(f32)
    e_alt = (r18 == 9 + 2 * c18).astype(f32)
    r5 = lax.broadcasted_iota(jnp.int32, (5, 5 * _P), 0)
    c5 = lax.broadcasted_iota(jnp.int32, (5, 5 * _P), 1)
    rep = (c5 // _P == r5).astype(f32)
    s50 =(lax.broadcasted_iota(jnp.int32, (5 * _P, 5), 0) // _P
           == lax.broadcasted_iota(jnp.int32, (5 * _P, 5), 1)).astype(f32)
    percs = (1.0 / (2.0 * _P)
             + (lax.broadcasted_iota(jnp.int32, (1, 5 * _P), 1) % _P
                ).astype(f32) / _P)
    ones_d = jnp.ones((_D, 1), f32)

    cate_col = x[:, 0:1]
    oh = (cate_col == lax.broadcasted_iota(jnp.int32, (1, 4), 1).astype(f32)
          ).astype(f32)
    cate_e = _mm(oh, cate_emb_ref[...])
    poi_e = half(poi_ref[...], par[:, 0:1])

    def softmax50(fv5):
        lg = -jnp.abs(_P * (_mm(fv5, rep) - percs))
        e = jnp.exp(lg)
        den = _mm(e, s50)
        return e * _mm(1.0 / den, rep)

    m_obs = softmax50(_mm(x, e_obs))
    m_alt = softmax50(_mm(x, e_alt))

    g_cate = _mm_t(perc_emb_ref[...], cate_emb_ref[...])
    gc_sel = _mm_t(oh, g_cate)
    gp = _mm_t(poi_e, perc_emb_ref[...])
    t_obs_c = _mm(m_obs * gc_sel, s50)
    t_obs_p = _mm(m_obs * gp, s50)
    t_alt_c = _mm(m_alt * gc_sel, s50)
    t_alt_p = _mm(m_alt * gp, s50)

    dots = []
    for j in range(6):
        c = half(cbg_ref[j], par[:, j + 1:j + 2])
        sgn = 1.0 if j == 0 else -1.0
        dots.append(_mm(cate_e * c, ones_d) * sgn)
        dots.append(_mm(poi_e * c, ones_d) * sgn)
    packed = jnp.concatenate(
        dots + [t_obs_c, t_obs_p, -t_alt_c, -t_alt_p], axis=1)

    col = lax.broadcasted_iota(jnp.int32, (1, 32), 1)
    w = jnp.where((col >= 2) & (col < 12), 0.2, 1.0)
    total = -jnp.sum(w * _log_sigmoid(packed))

    @pl.when(pl.program_id(0) == 0)
    def _init():
        out_ref[0, 0] = 0.0

    out_ref[0, 0] += total


_loss_tc = pl.pallas_call(
    _loss_body,
    grid=(_GRID,),
    in_specs=[
        pl.BlockSpec((_BLK, 18), lambda i: (i, 0)),
        pl.BlockSpec((4, _D), lambda i: (0, 0)),
        pl.BlockSpec((5 * _P, _D), lambda i: (0, 0)),
        pl.BlockSpec((_BLK, 2 * _D), lambda i: (i, 0)),
        pl.BlockSpec((6, _BLK, 2 * _D), lambda i: (0, i, 0)),
        pl.BlockSpec((_BLK, 8), lambda i: (i, 0)),
    ],
    out_specs=pl.BlockSpec(
        (1, 1), lambda i: (0, 0), memory_space=pltpu.SMEM),
    out_shape=jax.ShapeDtypeStruct((1, 1), jnp.float32),
    compiler_params=pltpu.CompilerParams(
        dimension_semantics=("arbitrary",)),
)


def kernel(inputs, cate_emb, poi_emb, cbg_emb, perc_emb):
    poi_ids = inputs[:, 1].astype(jnp.int32)
    cbg_cols = [2, 8, 10, 12, 14, 16]
    cbg_ids = jnp.concatenate(
        [inputs[:, c] for c in cbg_cols]).astype(jnp.int32)

    def rowof(i):
        return (i >> _PCS) * _PR + (i & (_PR - 1))

    def parof(i):
        return ((i >> (_PCS - 1)) & 1).astype(jnp.float32)

    poi_k, cbg_k = _gather_sc()
    cbg_pair = _make_pair(_CBG_GRID)(cbg_emb.T)
    (cbg_rows,) = cbg_k(cbg_pair, rowof(cbg_ids))
    poi_pair = _make_pair(_POI_GRID)(poi_emb.T)
    (poi_rows,) = poi_k(poi_pair, rowof(poi_ids))
    cbg_rows = cbg_rows.reshape(6, _B, 2 * _D)
    par = jnp.stack(
        [parof(poi_ids)] + [parof(cbg_ids[j * _B:(j + 1) * _B])
                            for j in range(6)]
        + [jnp.zeros((_B,), jnp.float32)], axis=1)
    out = _loss_tc(inputs, cate_emb, perc_emb, poi_rows, cbg_rows, par)
    return out[0, 0]


# speedup vs baseline: 1.0065x; 1.0017x over previous
import functools

import jax
import jax.numpy as jnp
from jax import lax
from jax.experimental import pallas as pl
from jax.experimental.pallas import tpu as pltpu
from jax.experimental.pallas import tpu_sc as plsc

_B = 16384
_P = 10
_D = 64
_NW = 32

_CBG_TOT = 6 * _B

_BLK = 2048
_GRID = _B // _BLK

_W = 512
_PC = 32768
_PR = _PC // 2
_PCS = _PC.bit_length() - 1

_CBG_GRID = (1000000 + _PC - 1) // _PC
_POI_GRID = (100000 + _PC - 1) // _PC


def _pair_body(tin_ref, out_ref):
    x = tin_ref[...]
    eye = (lax.broadcasted_iota(jnp.int32, (_D, _D), 0)
           == lax.broadcasted_iota(jnp.int32, (_D, _D), 1)).astype(jnp.float32)
    xt = jax.lax.dot_general(x, eye, (((0,), (0,)), ((), ())),
                             preferred_element_type=jnp.float32)
    out_ref[...] = jnp.concatenate([xt[:_PR], xt[_PR:]], axis=1)


def _make_pair(grid):
    return pl.pallas_call(
        _pair_body,
        grid=(grid,),
        in_specs=[pl.BlockSpec((_D, _PC), lambda i: (0, i))],
        out_specs=pl.BlockSpec((_PR, 2 * _D), lambda i: (i, 0)),
        out_shape=jax.ShapeDtypeStruct((grid * _PR, 2 * _D), jnp.float32),
        compiler_params=pltpu.CompilerParams(
            dimension_semantics=("parallel",)),
    )


def _sg_block(table, idx_hbm, out_hbm, base, idxv, rows, sem):
    pltpu.sync_copy(idx_hbm.at[pl.ds(base, _W)], idxv)
    pltpu.async_copy(table.at[idxv], rows, sem).wait()
    pltpu.sync_copy(rows, out_hbm.at[pl.ds(base, _W)])


def _gather_poi_body(poi_pair, poi_idx, poi_out, idxv, rows, sem):
    wid = lax.axis_index("s") * 2 + lax.axis_index("c")
    _sg_block(poi_pair, poi_idx, poi_out, wid * _W, idxv, rows, sem)


def _gather_cbg_body(cbg_pair, cbg_idx, cbg_out, idxv, rows, sem):
    wid = lax.axis_index("s") * 2 + lax.axis_index("c")
    for j in range(6):
        _sg_block(cbg_pair, cbg_idx, cbg_out, j * _B + wid * _W,
                  idxv, rows, sem)


@functools.cache
def _gather_sc():
    scratch = [
        pltpu.VMEM((_W,), jnp.int32),
        pltpu.VMEM((_W, 2 * _D), jnp.float32),
        pltpu.SemaphoreType.DMA,
    ]
    mesh = plsc.VectorSubcoreMesh(core_axis_name="c", subcore_axis_name="s")
    params = pltpu.CompilerParams(use_tc_tiling_on_sc=True)
    poi_k = pl.kernel(
        _gather_poi_body, mesh=mesh,
        out_type=[jax.ShapeDtypeStruct((_B, 2 * _D), jnp.float32)],
        scratch_types=scratch, compiler_params=params)
    cbg_k = pl.kernel(
        _gather_cbg_body, mesh=mesh,
        out_type=[jax.ShapeDtypeStruct((_CBG_TOT, 2 * _D), jnp.float32)],
        scratch_types=scratch, compiler_params=params)
    return poi_k, cbg_k


def _log_sigmoid(t):
    return jnp.minimum(t, 0.0) - jnp.log(1.0 + jnp.exp(-jnp.abs(t)))


def _mm(a, b):
    return jax.lax.dot_general(a, b, (((1,), (0,)), ((), ())),
                               preferred_element_type=jnp.float32)


def _mm_t(a, b):
    return jax.lax.dot_general(a, b, (((1,), (1,)), ((), ())),
                               preferred_element_type=jnp.float32)


def _loss_body(x_ref, cate_emb_ref, perc_emb_ref, poi_ref, cbg_ref, par_ref,
               out_ref):
    f32 = jnp.float32
    x = x_ref[...]
    par = par_ref[...]

    def half(xfull, p):
        return jnp.where(p > 0.5, xfull[:, _D:2 * _D], xfull[:, 0:_D])

    r18 = lax.broadcasted_iota(jnp.int32, (18, 5), 0)
    c18 = lax.broadcasted_iota(jnp.int32, (18, 5), 1)
    e_obs = (r18 == 3 + c18).astype(f32)
    e_alt = (r18 == 9 + 2 * c18).astype(f32)
    r5 = lax.broadcasted_iota(jnp.int32, (5, 5 * _P), 0)
    c5 = lax.broadcasted_iota(jnp.int32, (5, 5 * _P), 1)
    rep = (c5 // _P == r5).astype(f32)
    s50 =(lax.broadcasted_iota(jnp.int32, (5 * _P, 5), 0) // _P
           == lax.broadcasted_iota(jnp.int32, (5 * _P, 5), 1)).astype(f32)
    percs = (1.0 / (2.0 * _P)
             + (lax.broadcasted_iota(jnp.int32, (1, 5 * _P), 1) % _P
                ).astype(f32) / _P)
    ones_d = jnp.ones((_D, 1), f32)

    cate_col = x[:, 0:1]
    oh = (cate_col == lax.broadcasted_iota(jnp.int32, (1, 4), 1).astype(f32)
          ).astype(f32)
    cate_e = _mm(oh, cate_emb_ref[...])
    poi_e = half(poi_ref[...], par[:, 0:1])

    def softmax50(fv5):
        lg = -jnp.abs(_P * (_mm(fv5, rep) - percs))
        e = jnp.exp(lg)
        den = _mm(e, s50)
        return e * _mm(1.0 / den, rep)

    m_obs = softmax50(_mm(x, e_obs))
    m_alt = softmax50(_mm(x, e_alt))

    g_cate = _mm_t(perc_emb_ref[...], cate_emb_ref[...])
    gc_sel = _mm_t(oh, g_cate)
    gp = _mm_t(poi_e, perc_emb_ref[...])
    t_obs_c = _mm(m_obs * gc_sel, s50)
    t_obs_p = _mm(m_obs * gp, s50)
    t_alt_c = _mm(m_alt * gc_sel, s50)
    t_alt_p = _mm(m_alt * gp, s50)

    dots = []
    for j in range(6):
        c = half(cbg_ref[j], par[:, j + 1:j + 2])
        sgn = 1.0 if j == 0 else -1.0
        dots.append(_mm(cate_e * c, ones_d) * sgn)
        dots.append(_mm(poi_e * c, ones_d) * sgn)
    packed = jnp.concatenate(
        dots + [t_obs_c, t_obs_p, -t_alt_c, -t_alt_p], axis=1)

    col = lax.broadcasted_iota(jnp.int32, (1, 32), 1)
    w = jnp.where((col >= 2) & (col < 12), 0.2, 1.0)
    total = -jnp.sum(w * _log_sigmoid(packed))

    @pl.when(pl.program_id(0) == 0)
    def _init():
        out_ref[0, 0] = 0.0

    out_ref[0, 0] += total


_loss_tc = pl.pallas_call(
    _loss_body,
    grid=(_GRID,),
    in_specs=[
        pl.BlockSpec((_BLK, 18), lambda i: (i, 0)),
        pl.BlockSpec((4, _D), lambda i: (0, 0)),
        pl.BlockSpec((5 * _P, _D), lambda i: (0, 0)),
        pl.BlockSpec((_BLK, 2 * _D), lambda i: (i, 0)),
        pl.BlockSpec((6, _BLK, 2 * _D), lambda i: (0, i, 0)),
        pl.BlockSpec((_BLK, 8), lambda i: (i, 0)),
    ],
    out_specs=pl.BlockSpec(
        (1, 1), lambda i: (0, 0), memory_space=pltpu.SMEM),
    out_shape=jax.ShapeDtypeStruct((1, 1), jnp.float32),
    compiler_params=pltpu.CompilerParams(
        dimension_semantics=("arbitrary",)),
)


def kernel(inputs, cate_emb, poi_emb, cbg_emb, perc_emb):
    poi_ids = inputs[:, 1].astype(jnp.int32)
    cbg_cols = [2, 8, 10, 12, 14, 16]
    cbg_ids = jnp.concatenate(
        [inputs[:, c] for c in cbg_cols]).astype(jnp.int32)

    def rowof(i):
        return (i >> _PCS) * _PR + (i & (_PR - 1))

    def parof(i):
        return ((i >> (_PCS - 1)) & 1).astype(jnp.float32)

    poi_k, cbg_k = _gather_sc()
    cbg_pair = _make_pair(_CBG_GRID)(cbg_emb.T)
    (cbg_rows,) = cbg_k(cbg_pair, rowof(cbg_ids))
    poi_pair = _make_pair(_POI_GRID)(poi_emb.T)
    (poi_rows,) = poi_k(poi_pair, rowof(poi_ids))
    cbg_rows = cbg_rows.reshape(6, _B, 2 * _D)
    par = jnp.stack(
        [parof(poi_ids)] + [parof(cbg_ids[j * _B:(j + 1) * _B])
                            for j in range(6)]
        + [jnp.zeros((_B,), jnp.float32)], axis=1)
    out = _loss_tc(inputs, cate_emb, perc_emb, poi_rows, cbg_rows, par)
    return out[0, 0]


# speedup vs baseline: 1.0853x; 1.0782x over previous
import functools

import jax
import jax.numpy as jnp
from jax import lax
from jax.experimental import pallas as pl
from jax.experimental.pallas import tpu as pltpu
from jax.experimental.pallas import tpu_sc as plsc

_B = 16384
_P = 10
_D = 64
_NW = 32

_CBG_TOT = 6 * _B

_BLK = 2048
_GRID = _B // _BLK

_W = 512
_PC = 32768
_PR = _PC // 2
_PCS = _PC.bit_length() - 1

_CBG_GRID = (1000000 + _PC - 1) // _PC
_POI_GRID = (100000 + _PC - 1) // _PC


def _pair_body(tin_ref, out_ref):
    x = tin_ref[...]
    eye = (lax.broadcasted_iota(jnp.int32, (_D, _D), 0)
           == lax.broadcasted_iota(jnp.int32, (_D, _D), 1)).astype(jnp.float32)
    xt = jax.lax.dot_general(x, eye, (((0,), (0,)), ((), ())),
                             preferred_element_type=jnp.float32)
    out_ref[...] = jnp.concatenate([xt[:_PR], xt[_PR:]], axis=1)


def _make_pair(grid):
    return pl.pallas_call(
        _pair_body,
        grid=(grid,),
        in_specs=[pl.BlockSpec((_D, _PC), lambda i: (0, i))],
        out_specs=pl.BlockSpec((_PR, 2 * _D), lambda i: (i, 0)),
        out_shape=jax.ShapeDtypeStruct((grid * _PR, 2 * _D), jnp.float32),
        compiler_params=pltpu.CompilerParams(
            dimension_semantics=("parallel",)),
    )


def _sg_block(table, idx_hbm, out_hbm, base, idxv, rows, sem):
    pltpu.sync_copy(idx_hbm.at[pl.ds(base, _W)], idxv)
    pltpu.async_copy(table.at[idxv], rows, sem).wait()
    pltpu.sync_copy(rows, out_hbm.at[pl.ds(base, _W)])


def _gather_poi_body(poi_pair, poi_idx, poi_out, idxv, rows, sem):
    wid = lax.axis_index("s") * 2 + lax.axis_index("c")
    _sg_block(poi_pair, poi_idx, poi_out, wid * _W, idxv, rows, sem)


def _gather_cbg_body(cbg_pair, cbg_idx, cbg_out, idxv, rows, sem):
    wid = lax.axis_index("s") * 2 + lax.axis_index("c")
    for j in range(6):
        _sg_block(cbg_pair, cbg_idx, cbg_out, j * _B + wid * _W,
                  idxv, rows, sem)


@functools.cache
def _gather_sc():
    scratch = [
        pltpu.VMEM((_W,), jnp.int32),
        pltpu.VMEM((_W, 2 * _D), jnp.float32),
        pltpu.SemaphoreType.DMA,
    ]
    mesh = plsc.VectorSubcoreMesh(core_axis_name="c", subcore_axis_name="s")
    params = pltpu.CompilerParams(use_tc_tiling_on_sc=True)
    poi_k = pl.kernel(
        _gather_poi_body, mesh=mesh,
        out_type=[jax.ShapeDtypeStruct((_B, 2 * _D), jnp.float32)],
        scratch_types=scratch, compiler_params=params)
    cbg_k = pl.kernel(
        _gather_cbg_body, mesh=mesh,
        out_type=[jax.ShapeDtypeStruct((_CBG_TOT, 2 * _D), jnp.float32)],
        scratch_types=scratch, compiler_params=params)
    return poi_k, cbg_k


def _log_sigmoid(t):
    return jnp.minimum(t, 0.0) - jnp.log(1.0 + jnp.exp(-jnp.abs(t)))


def _mm(a, b):
    return jax.lax.dot_general(a, b, (((1,), (0,)), ((), ())),
                               preferred_element_type=jnp.float32)


def _mm_t(a, b):
    return jax.lax.dot_general(a, b, (((1,), (1,)), ((), ())),
                               preferred_element_type=jnp.float32)


def _loss_body(x_ref, cate_emb_ref, perc_emb_ref, poi_ref, cbg_ref, par_ref,
               out_ref):
    f32 = jnp.float32
    x = x_ref[...]
    par = par_ref[...]

    def half(xfull, p):
        return jnp.where(p > 0.5, xfull[:, _D:2 * _D], xfull[:, 0:_D])

    r18 = lax.broadcasted_iota(jnp.int32, (18, 5), 0)
    c18 = lax.broadcasted_iota(jnp.int32, (18, 5), 1)
    e_obs = (r18 == 3 + c18).astype(f32)
    e_alt = (r18 == 9 + 2 * c18).astype(f32)
    r5 = lax.broadcasted_iota(jnp.int32, (5, 5 * _P), 0)
    c5 = lax.broadcasted_iota(jnp.int32, (5, 5 * _P), 1)
    rep = (c5 // _P == r5).astype(f32)
    s50 =(lax.broadcasted_iota(jnp.int32, (5 * _P, 5), 0) // _P
           == lax.broadcasted_iota(jnp.int32, (5 * _P, 5), 1)).astype(f32)
    percs = (1.0 / (2.0 * _P)
             + (lax.broadcasted_iota(jnp.int32, (1, 5 * _P), 1) % _P
                ).astype(f32) / _P)
    ones_d = jnp.ones((_D, 1), f32)

    cate_col = x[:, 0:1]
    oh = (cate_col == lax.broadcasted_iota(jnp.int32, (1, 4), 1).astype(f32)
          ).astype(f32)
    cate_e = _mm(oh, cate_emb_ref[...])
    poi_e = half(poi_ref[...], par[:, 0:1])

    def softmax50(fv5):
        lg = -jnp.abs(_P * (_mm(fv5, rep) - percs))
        e = jnp.exp(lg)
        den = _mm(e, s50)
        return e * _mm(1.0 / den, rep)

    m_obs = softmax50(_mm(x, e_obs))
    m_alt = softmax50(_mm(x, e_alt))

    g_cate = _mm_t(perc_emb_ref[...], cate_emb_ref[...])
    gc_sel = _mm_t(oh, g_cate)
    gp = _mm_t(poi_e, perc_emb_ref[...])
    t_obs_c = _mm(m_obs * gc_sel, s50)
    t_obs_p = _mm(m_obs * gp, s50)
    t_alt_c = _mm(m_alt * gc_sel, s50)
    t_alt_p = _mm(m_alt * gp, s50)

    bd = (lax.broadcasted_iota(jnp.int32, (6 * _D, 6), 0) // _D
          == lax.broadcasted_iota(jnp.int32, (6 * _D, 6), 1)).astype(f32)
    cs = [half(cbg_ref[j], par[:, j + 1:j + 2]) for j in range(6)]
    prod_c = jnp.concatenate([cate_e * c for c in cs], axis=1)
    prod_p = jnp.concatenate([poi_e * c for c in cs], axis=1)
    d_c = _mm(prod_c, bd)
    d_p = _mm(prod_p, bd)
    packed = jnp.concatenate(
        [d_c, d_p, t_obs_c, t_obs_p, -t_alt_c, -t_alt_p], axis=1)

    col = lax.broadcasted_iota(jnp.int32, (1, 32), 1)
    neg = (col % 6 > 0) & (col < 12)
    sgn = jnp.where(neg, -1.0, 1.0)
    w = jnp.where(neg, 0.2, 1.0)
    total = -jnp.sum(w * _log_sigmoid(sgn * packed))

    @pl.when(pl.program_id(0) == 0)
    def _init():
        out_ref[0, 0] = 0.0

    out_ref[0, 0] += total


_loss_tc = pl.pallas_call(
    _loss_body,
    grid=(_GRID,),
    in_specs=[
        pl.BlockSpec((_BLK, 18), lambda i: (i, 0)),
        pl.BlockSpec((4, _D), lambda i: (0, 0)),
        pl.BlockSpec((5 * _P, _D), lambda i: (0, 0)),
        pl.BlockSpec((_BLK, 2 * _D), lambda i: (i, 0)),
        pl.BlockSpec((6, _BLK, 2 * _D), lambda i: (0, i, 0)),
        pl.BlockSpec((_BLK, 8), lambda i: (i, 0)),
    ],
    out_specs=pl.BlockSpec(
        (1, 1), lambda i: (0, 0), memory_space=pltpu.SMEM),
    out_shape=jax.ShapeDtypeStruct((1, 1), jnp.float32),
    compiler_params=pltpu.CompilerParams(
        dimension_semantics=("arbitrary",)),
)


def kernel(inputs, cate_emb, poi_emb, cbg_emb, perc_emb):
    poi_ids = inputs[:, 1].astype(jnp.int32)
    cbg_cols = [2, 8, 10, 12, 14, 16]
    cbg_ids = jnp.concatenate(
        [inputs[:, c] for c in cbg_cols]).astype(jnp.int32)

    def rowof(i):
        return (i >> _PCS) * _PR + (i & (_PR - 1))

    def parof(i):
        return ((i >> (_PCS - 1)) & 1).astype(jnp.float32)

    poi_k, cbg_k = _gather_sc()
    cbg_pair = _make_pair(_CBG_GRID)(cbg_emb.T)
    (cbg_rows,) = cbg_k(cbg_pair, rowof(cbg_ids))
    poi_pair = _make_pair(_POI_GRID)(poi_emb.T)
    (poi_rows,) = poi_k(poi_pair, rowof(poi_ids))
    cbg_rows = cbg_rows.reshape(6, _B, 2 * _D)
    par = jnp.stack(
        [parof(poi_ids)] + [parof(cbg_ids[j * _B:(j + 1) * _B])
                            for j in range(6)]
        + [jnp.zeros((_B,), jnp.float32)], axis=1)
    out = _loss_tc(inputs, cate_emb, perc_emb, poi_rows, cbg_rows, par)
    return out[0, 0]
